# baseline (speedup 1.0000x reference)
import functools
import jax
import jax.numpy as jnp
import numpy as np
from jax import lax
from jax.experimental import pallas as pl
from jax.experimental.pallas import tpu as pltpu
from jax.experimental.pallas import tpu_sc as plsc

BN = float(1.0 / np.sqrt(1.0 + 1e-5))
F32 = jnp.float32
I32 = jnp.int32


def _fps_body(B, npoint, N, xyzP_ref, xyzR_ref, xyzS_ref, newR_ref):
    NL = N // 8
    iota = (lax.broadcasted_iota(I32, (8, NL), 0) * NL
            + lax.broadcasted_iota(I32, (8, NL), 1))

    def step(s, carry):
        for b in range(B):
            newR_ref[b, pl.ds(s, 1), :] = xyzR_ref[b, pl.ds(carry[b][1], 1), :]
        nd = []
        for b in range(B):
            base = (b * N + carry[b][1]) * 3
            dx = xyzP_ref[b, 0] - xyzS_ref[base]
            dy = xyzP_ref[b, 1] - xyzS_ref[base + 1]
            dz = xyzP_ref[b, 2] - xyzS_ref[base + 2]
            d = dx * dx + dy * dy + dz * dz
            nd.append(jnp.minimum(carry[b][0], d))
        ms = [jnp.max(nd[b]) for b in range(B)]
        return tuple(
            (nd[b],
             jnp.min(jnp.where(nd[b] == ms[b], iota, N)).astype(I32))
            for b in range(B))

    init = tuple((jnp.full((8, NL), 1e10, F32), jnp.int32(0)) for _ in range(B))
    lax.fori_loop(0, npoint, step, init)


def _fps(xyzP, xyzR, xyzS, npoint):
    B, _, _, NL = xyzP.shape
    N = 8 * NL
    return pl.pallas_call(
        functools.partial(_fps_body, B, npoint, N),
        in_specs=[
            pl.BlockSpec((B, 3, 8, NL), lambda: (0, 0, 0, 0)),
            pl.BlockSpec((B, N, 8), lambda: (0, 0, 0)),
            pl.BlockSpec(memory_space=pltpu.SMEM),
        ],
        out_specs=pl.BlockSpec((B, npoint, 8), lambda: (0, 0, 0)),
        out_shape=jax.ShapeDtypeStruct((B, npoint, 8), F32),
    )(xyzP, xyzR, xyzS)


def _pw_body(xyzR_ref, ptsR_ref, wa_ref, wb_ref, pw_ref):
    pw = jnp.dot(xyzR_ref[0], wa_ref[...], preferred_element_type=F32)
    pw = pw + jnp.dot(ptsR_ref[0], wb_ref[...], preferred_element_type=F32)
    pw_ref[0] = pw


def _pw(xyzR, ptsR, waT, wbT):
    B, N, _ = xyzR.shape
    C1 = waT.shape[1]
    Cp = ptsR.shape[2]
    return pl.pallas_call(
        _pw_body,
        grid=(B,),
        in_specs=[
            pl.BlockSpec((1, N, 8), lambda b: (b, 0, 0)),
            pl.BlockSpec((1, N, Cp), lambda b: (b, 0, 0)),
            pl.BlockSpec((8, C1), lambda b: (0, 0)),
            pl.BlockSpec((Cp, C1), lambda b: (0, 0)),
        ],
        out_specs=pl.BlockSpec((1, N, C1), lambda b: (b, 0, 0)),
        out_shape=jax.ShapeDtypeStruct((B, N, C1), F32),
    )(xyzR, ptsR, waT, wbT)


def _bq_body(r2, nsample, N, xyzR_ref, newT_ref, gidx_ref):
    b = pl.program_id(0)
    P = xyzR_ref[0]
    Q = newT_ref[0]
    Sb = Q.shape[1]
    pn = jnp.sum(P * P, axis=1, keepdims=True)
    qn = jnp.sum(Q * Q, axis=0, keepdims=True)
    sqrT = (qn + pn) - 2.0 * jnp.dot(P, Q, preferred_element_type=F32)
    iotaN = lax.broadcasted_iota(I32, (N, Sb), 0)
    candbase = jnp.where(sqrT <= r2, iotaN, N)
    last = jnp.full((1, Sb), -1, I32)
    first = None
    off = b * N
    for k in range(nsample):
        cand = jnp.where(candbase > last, candbase, N)
        idx_k = jnp.min(cand, axis=0, keepdims=True)
        last = idx_k
        if k == 0:
            first = idx_k
        gidx_ref[0, k:k + 1, :] = jnp.where(idx_k == N, first, idx_k) + off


def _bq(xyzR, newT, radius, nsample):
    B, N, _ = xyzR.shape
    S = newT.shape[2]
    Sb = min(S, 128)
    return pl.pallas_call(
        functools.partial(_bq_body, float(radius) * float(radius), nsample, N),
        grid=(B, S // Sb),
        in_specs=[
            pl.BlockSpec((1, N, 8), lambda b, j: (b, 0, 0)),
            pl.BlockSpec((1, 8, Sb), lambda b, j: (b, 0, j)),
        ],
        out_specs=pl.BlockSpec((1, nsample, Sb), lambda b, j: (b, 0, j)),
        out_shape=jax.ShapeDtypeStruct((B, nsample, S), I32),
    )(xyzR, newT)


def _sc_gather(table, idx):
    R, C = table.shape
    M = idx.shape[0]
    info = plsc.get_sparse_core_info()
    NC, NS = info.num_cores, info.num_subcores
    NW = NC * NS
    b_per_w = M // NW
    c_rows = min(b_per_w, max(8, 65536 // C))
    while b_per_w % c_rows:
        c_rows //= 2
    nchunks = b_per_w // c_rows
    mesh = plsc.VectorSubcoreMesh(core_axis_name="c", subcore_axis_name="s")

    @functools.partial(
        pl.kernel,
        out_type=jax.ShapeDtypeStruct((M, C), F32),
        mesh=mesh,
        compiler_params=pltpu.CompilerParams(use_tc_tiling_on_sc=False),
        scratch_types=[
            pltpu.VMEM((c_rows,), I32),
            pltpu.VMEM((c_rows, C), F32),
            pltpu.SemaphoreType.DMA,
        ],
    )
    def k(table_hbm, idx_hbm, out_hbm, idx_v, rows_v, sem):
        wid = lax.axis_index("s") * NC + lax.axis_index("c")
        for ch in range(nchunks):
            base = wid * b_per_w + ch * c_rows
            pltpu.sync_copy(idx_hbm.at[pl.ds(base, c_rows)], idx_v)
            pltpu.async_copy(table_hbm.at[idx_v], rows_v, sem).wait()
            pltpu.sync_copy(rows_v, out_hbm.at[pl.ds(base, c_rows)])

    return k(table, idx)


def _mlp_body(K, G_ref, newR_ref, wa_ref, g0_ref, b0_ref, w1_ref, g1_ref,
              b1_ref, w2_ref, g2_ref, b2_ref, out_ref):
    G = G_ref[0]
    Sb = G.shape[1]
    C1 = G.shape[2]
    Qc = jnp.dot(newR_ref[0], wa_ref[...], preferred_element_type=F32)
    h = G - Qc.reshape(1, Sb, C1)
    h = jax.nn.relu(h * (g0_ref[...].reshape(1, 1, C1) * BN)
                    + b0_ref[...].reshape(1, 1, C1))
    h = h.reshape(K * Sb, C1)
    h = jax.nn.relu(jnp.dot(h, w1_ref[...], preferred_element_type=F32)
                    * (g1_ref[...] * BN) + b1_ref[...])
    h = jax.nn.relu(jnp.dot(h, w2_ref[...], preferred_element_type=F32)
                    * (g2_ref[...] * BN) + b2_ref[...])
    C2 = h.shape[1]
    out_ref[0] = jnp.max(h.reshape(K, Sb, C2), axis=0)


def _mlp(G, newR, waT, g0, b0, w1T, g1, b1, w2T, g2, b2):
    B, K, S, C1 = G.shape
    C1b = w1T.shape[1]
    C2 = w2T.shape[1]
    Sb = min(S, 256)
    return pl.pallas_call(
        functools.partial(_mlp_body, K),
        grid=(B, S // Sb),
        in_specs=[
            pl.BlockSpec((1, K, Sb, C1), lambda b, j: (b, 0, j, 0)),
            pl.BlockSpec((1, Sb, 8), lambda b, j: (b, j, 0)),
            pl.BlockSpec((8, C1), lambda b, j: (0, 0)),
            pl.BlockSpec((1, C1), lambda b, j: (0, 0)),
            pl.BlockSpec((1, C1), lambda b, j: (0, 0)),
            pl.BlockSpec((C1, C1b), lambda b, j: (0, 0)),
            pl.BlockSpec((1, C1b), lambda b, j: (0, 0)),
            pl.BlockSpec((1, C1b), lambda b, j: (0, 0)),
            pl.BlockSpec((C1b, C2), lambda b, j: (0, 0)),
            pl.BlockSpec((1, C2), lambda b, j: (0, 0)),
            pl.BlockSpec((1, C2), lambda b, j: (0, 0)),
        ],
        out_specs=pl.BlockSpec((1, Sb, C2), lambda b, j: (b, j, 0)),
        out_shape=jax.ShapeDtypeStruct((B, S, C2), F32),
    )(G, newR, waT, g0.reshape(1, -1), b0.reshape(1, -1), w1T,
      g1.reshape(1, -1), b1.reshape(1, -1), w2T, g2.reshape(1, -1),
      b2.reshape(1, -1))


def _fp_body(S2, nl, x1_ref, x2_ref, p1_ref, p2_ref, *refs):
    wa_ref, wb_ref = refs[0], refs[1]
    lrefs = refs[2:-1]
    out_ref = refs[-1]
    q1 = x1_ref[0]
    q2 = x2_ref[0]
    Sb = q1.shape[0]
    qn1 = jnp.sum(q1 * q1, axis=1, keepdims=True)
    qn2 = jnp.sum(q2 * q2, axis=0, keepdims=True)
    d = (qn1 + qn2) - 2.0 * jnp.dot(q1, q2, preferred_element_type=F32)
    iota = lax.broadcasted_iota(I32, (Sb, S2), 1)
    ws, idxs = [], []
    for _ in range(3):
        dk = jnp.min(d, axis=1, keepdims=True)
        ik = jnp.min(jnp.where(d == dk, iota, S2), axis=1, keepdims=True)
        d = jnp.where(iota == ik, jnp.float32(jnp.inf), d)
        ws.append(1.0 / (jnp.maximum(dk, 0.0) + 1e-8))
        idxs.append(ik)
    wsum = (ws[0] + ws[1]) + ws[2]
    wfull = jnp.where(iota == idxs[0], ws[0] / wsum, 0.0)
    wfull = wfull + jnp.where(iota == idxs[1], ws[1] / wsum, 0.0)
    wfull = wfull + jnp.where(iota == idxs[2], ws[2] / wsum, 0.0)
    T2 = jnp.dot(p2_ref[0], wb_ref[...], preferred_element_type=F32)
    pre = jnp.dot(wfull, T2, preferred_element_type=F32)
    pre = pre + jnp.dot(p1_ref[0], wa_ref[...], preferred_element_type=F32)
    g0, b0 = lrefs[0], lrefs[1]
    h = jax.nn.relu(pre * (g0[...] * BN) + b0[...])
    i = 2
    for _ in range(nl - 1):
        w_, g_, b_ = lrefs[i], lrefs[i + 1], lrefs[i + 2]
        h = jax.nn.relu(jnp.dot(h, w_[...], preferred_element_type=F32)
                        * (g_[...] * BN) + b_[...])
        i += 3
    if len(lrefs) > i:
        w_, g_, b_ = lrefs[i], lrefs[i + 1], lrefs[i + 2]
        h = jax.nn.relu(jnp.dot(h, w_[...], preferred_element_type=F32)
                        * (g_[...] * BN) + b_[...])
        w_, b_ = lrefs[i + 3], lrefs[i + 4]
        h = jnp.dot(h, w_[...], preferred_element_type=F32) + b_[...]
    out_ref[0] = h


def _fp(x1R, x2T, p1R, p2R, waT, wbT, layers, fc=None):
    B, S1, _ = x1R.shape
    S2 = x2T.shape[2]
    C1p = p1R.shape[2]
    C2p = p2R.shape[2]
    Sb = min(S1, 512)
    nl = len(layers)
    flat = [layers[0][1].reshape(1, -1), layers[0][2].reshape(1, -1)]
    for (wT, g, b) in layers[1:]:
        flat += [wT, g.reshape(1, -1), b.reshape(1, -1)]
    if fc is not None:
        fc1wT, fc1g, fc1b, fc2wT, fc2bias = fc
        flat += [fc1wT, fc1g.reshape(1, -1), fc1b.reshape(1, -1), fc2wT,
                 fc2bias.reshape(1, -1)]
    especs = [pl.BlockSpec(a.shape, lambda b, j, n=a.ndim: (0,) * n)
              for a in flat]
    Cout = fc[3].shape[1] if fc is not None else layers[-1][0].shape[1]
    return pl.pallas_call(
        functools.partial(_fp_body, S2, nl),
        grid=(B, S1 // Sb),
        in_specs=[
            pl.BlockSpec((1, Sb, 8), lambda b, j: (b, j, 0)),
            pl.BlockSpec((1, 8, S2), lambda b, j: (b, 0, 0)),
            pl.BlockSpec((1, Sb, C1p), lambda b, j: (b, j, 0)),
            pl.BlockSpec((1, S2, C2p), lambda b, j: (b, 0, 0)),
            pl.BlockSpec(waT.shape, lambda b, j: (0, 0)),
            pl.BlockSpec(wbT.shape, lambda b, j: (0, 0)),
        ] + especs,
        out_specs=pl.BlockSpec((1, Sb, Cout), lambda b, j: (b, j, 0)),
        out_shape=jax.ShapeDtypeStruct((B, S1, Cout), F32),
    )(x1R, x2T, p1R, p2R, waT, wbT, *flat)


def _sa_feat(xyzR, gidx, newR, ptsR, params, prefix, nsample):
    B, N, _ = xyzR.shape
    npoint = newR.shape[1]
    w0 = params[prefix + '_w0']
    C1 = w0.shape[0]
    waT = jnp.zeros((8, C1), F32).at[:3].set(w0[:, :3].T)
    wbT = w0[:, 3:].T
    PW = _pw(xyzR, ptsR, waT, wbT)
    G = _sc_gather(PW.reshape(B * N, C1), gidx.reshape(-1))
    G = G.reshape(B, nsample, npoint, C1)
    return _mlp(G, newR, waT, params[prefix + '_g0'], params[prefix + '_b0'],
                params[prefix + '_w1'].T, params[prefix + '_g1'],
                params[prefix + '_b1'], params[prefix + '_w2'].T,
                params[prefix + '_g2'], params[prefix + '_b2'])


def _fp_level(x1R, x2T, p1R, p2R, params, prefix, nlayers, fc=None):
    C1p = p1R.shape[2]
    w0 = params[prefix + '_w0']
    waT = w0[:, :C1p].T
    if C1p < 8:
        waT = jnp.zeros((8, w0.shape[0]), F32).at[:C1p].set(waT)
        B, S1, _ = p1R.shape
        p1R = jnp.concatenate([p1R, jnp.zeros((B, S1, 8 - C1p), F32)], axis=-1)
    wbT = w0[:, C1p:].T
    layers = [(waT, params[prefix + '_g0'], params[prefix + '_b0'])]
    for j in range(1, nlayers):
        layers.append((params[prefix + '_w' + str(j)].T,
                       params[prefix + '_g' + str(j)],
                       params[prefix + '_b' + str(j)]))
    return _fp(x1R, x2T, p1R, p2R, waT, wbT, layers, fc=fc)


def kernel(xyz, points, params):
    B, N, _ = xyz.shape
    SA = (('sa0', 1024, 0.1), ('sa1', 256, 0.2), ('sa2', 64, 0.4),
          ('sa3', 16, 0.8))
    xyzR = jnp.concatenate([xyz, jnp.zeros((B, N, 5), F32)], axis=-1)
    xs = [xyzR]
    gs = []
    cur = xyzR
    for (prefix, npoint, radius) in SA:
        n = cur.shape[1]
        curT = jnp.transpose(cur, (0, 2, 1))
        newR = _fps(curT[:, :3, :].reshape(B, 3, 8, n // 8), cur,
                    cur[:, :, :3].reshape(-1), npoint)
        gs.append(_bq(cur, jnp.transpose(newR, (0, 2, 1)), radius, 32))
        xs.append(newR)
        cur = newR
    feats = [points]
    for i, (prefix, npoint, radius) in enumerate(SA):
        feats.append(_sa_feat(xs[i], gs[i], xs[i + 1], feats[i], params,
                              prefix, 32))
    l1p, l2p, l3p, l4p = feats[1], feats[2], feats[3], feats[4]
    x1T = jnp.transpose(xs[1], (0, 2, 1))
    x2T = jnp.transpose(xs[2], (0, 2, 1))
    x3T = jnp.transpose(xs[3], (0, 2, 1))
    x4T = jnp.transpose(xs[4], (0, 2, 1))
    l3p = _fp_level(xs[3], x4T, l3p, l4p, params, 'fp0', 2)
    l2p = _fp_level(xs[2], x3T, l2p, l3p, params, 'fp1', 2)
    l1p = _fp_level(xs[1], x2T, l1p, l2p, params, 'fp2', 2)
    fc = (params['fc1_w'].T, params['fc1_g'], params['fc1_b'],
          params['fc2_w'].T, params['fc2_bias'])
    return _fp_level(xyzR, x1T, points, l1p, params, 'fp3', 3, fc=fc)


# speedup vs baseline: 1.0049x; 1.0049x over previous
import functools
import jax
import jax.numpy as jnp
import numpy as np
from jax import lax
from jax.experimental import pallas as pl
from jax.experimental.pallas import tpu as pltpu
from jax.experimental.pallas import tpu_sc as plsc

BN = float(1.0 / np.sqrt(1.0 + 1e-5))
F32 = jnp.float32
I32 = jnp.int32


def _fps_body(B, npoint, N, xyzP_ref, xyzR_ref, xyzS_ref, newR_ref):
    NL = N // 8
    iota = (lax.broadcasted_iota(I32, (8, NL), 0) * NL
            + lax.broadcasted_iota(I32, (8, NL), 1))

    def step(s, carry):
        for b in range(B):
            newR_ref[b, pl.ds(s, 1), :] = xyzR_ref[b, pl.ds(carry[b][1], 1), :]
        nd = []
        for b in range(B):
            base = (b * N + carry[b][1]) * 3
            dx = xyzP_ref[b, 0] - xyzS_ref[base]
            dy = xyzP_ref[b, 1] - xyzS_ref[base + 1]
            dz = xyzP_ref[b, 2] - xyzS_ref[base + 2]
            d = dx * dx + dy * dy + dz * dz
            nd.append(jnp.minimum(carry[b][0], d))
        ms = [jnp.max(nd[b]) for b in range(B)]
        return tuple(
            (nd[b],
             jnp.min(jnp.where(nd[b] == ms[b], iota, N)).astype(I32))
            for b in range(B))

    init = tuple((jnp.full((8, NL), 1e10, F32), jnp.int32(0)) for _ in range(B))
    lax.fori_loop(0, npoint, step, init, unroll=2)


def _fps(xyzP, xyzR, xyzS, npoint):
    B, _, _, NL = xyzP.shape
    N = 8 * NL
    return pl.pallas_call(
        functools.partial(_fps_body, B, npoint, N),
        in_specs=[
            pl.BlockSpec((B, 3, 8, NL), lambda: (0, 0, 0, 0)),
            pl.BlockSpec((B, N, 8), lambda: (0, 0, 0)),
            pl.BlockSpec(memory_space=pltpu.SMEM),
        ],
        out_specs=pl.BlockSpec((B, npoint, 8), lambda: (0, 0, 0)),
        out_shape=jax.ShapeDtypeStruct((B, npoint, 8), F32),
    )(xyzP, xyzR, xyzS)


def _pw_body(xyzR_ref, ptsR_ref, wa_ref, wb_ref, pw_ref):
    pw = jnp.dot(xyzR_ref[0], wa_ref[...], preferred_element_type=F32)
    pw = pw + jnp.dot(ptsR_ref[0], wb_ref[...], preferred_element_type=F32)
    pw_ref[0] = pw


def _pw(xyzR, ptsR, waT, wbT):
    B, N, _ = xyzR.shape
    C1 = waT.shape[1]
    Cp = ptsR.shape[2]
    return pl.pallas_call(
        _pw_body,
        grid=(B,),
        in_specs=[
            pl.BlockSpec((1, N, 8), lambda b: (b, 0, 0)),
            pl.BlockSpec((1, N, Cp), lambda b: (b, 0, 0)),
            pl.BlockSpec((8, C1), lambda b: (0, 0)),
            pl.BlockSpec((Cp, C1), lambda b: (0, 0)),
        ],
        out_specs=pl.BlockSpec((1, N, C1), lambda b: (b, 0, 0)),
        out_shape=jax.ShapeDtypeStruct((B, N, C1), F32),
    )(xyzR, ptsR, waT, wbT)


def _bq_body(r2, nsample, N, xyzR_ref, newT_ref, gidx_ref):
    b = pl.program_id(0)
    P = xyzR_ref[0]
    Q = newT_ref[0]
    Sb = Q.shape[1]
    pn = jnp.sum(P * P, axis=1, keepdims=True)
    qn = jnp.sum(Q * Q, axis=0, keepdims=True)
    sqrT = (qn + pn) - 2.0 * jnp.dot(P, Q, preferred_element_type=F32)
    iotaN = lax.broadcasted_iota(I32, (N, Sb), 0)
    candbase = jnp.where(sqrT <= r2, iotaN, N)
    last = jnp.full((1, Sb), -1, I32)
    first = None
    off = b * N
    for k in range(nsample):
        cand = jnp.where(candbase > last, candbase, N)
        idx_k = jnp.min(cand, axis=0, keepdims=True)
        last = idx_k
        if k == 0:
            first = idx_k
        gidx_ref[0, k:k + 1, :] = jnp.where(idx_k == N, first, idx_k) + off


def _bq(xyzR, newT, radius, nsample):
    B, N, _ = xyzR.shape
    S = newT.shape[2]
    Sb = min(S, 256)
    return pl.pallas_call(
        functools.partial(_bq_body, float(radius) * float(radius), nsample, N),
        grid=(B, S // Sb),
        in_specs=[
            pl.BlockSpec((1, N, 8), lambda b, j: (b, 0, 0)),
            pl.BlockSpec((1, 8, Sb), lambda b, j: (b, 0, j)),
        ],
        out_specs=pl.BlockSpec((1, nsample, Sb), lambda b, j: (b, 0, j)),
        out_shape=jax.ShapeDtypeStruct((B, nsample, S), I32),
    )(xyzR, newT)


def _sc_gather(table, idx):
    R, C = table.shape
    M = idx.shape[0]
    info = plsc.get_sparse_core_info()
    NC, NS = info.num_cores, info.num_subcores
    NW = NC * NS
    b_per_w = M // NW
    c_rows = min(b_per_w, max(8, 65536 // C))
    while b_per_w % c_rows:
        c_rows //= 2
    nchunks = b_per_w // c_rows
    mesh = plsc.VectorSubcoreMesh(core_axis_name="c", subcore_axis_name="s")

    @functools.partial(
        pl.kernel,
        out_type=jax.ShapeDtypeStruct((M, C), F32),
        mesh=mesh,
        compiler_params=pltpu.CompilerParams(use_tc_tiling_on_sc=False),
        scratch_types=[
            pltpu.VMEM((c_rows,), I32),
            pltpu.VMEM((c_rows, C), F32),
            pltpu.SemaphoreType.DMA,
        ],
    )
    def k(table_hbm, idx_hbm, out_hbm, idx_v, rows_v, sem):
        wid = lax.axis_index("s") * NC + lax.axis_index("c")
        for ch in range(nchunks):
            base = wid * b_per_w + ch * c_rows
            pltpu.sync_copy(idx_hbm.at[pl.ds(base, c_rows)], idx_v)
            pltpu.async_copy(table_hbm.at[idx_v], rows_v, sem).wait()
            pltpu.sync_copy(rows_v, out_hbm.at[pl.ds(base, c_rows)])

    return k(table, idx)


def _mlp_body(K, G_ref, newR_ref, wa_ref, g0_ref, b0_ref, w1_ref, g1_ref,
              b1_ref, w2_ref, g2_ref, b2_ref, out_ref):
    G = G_ref[0]
    Sb = G.shape[1]
    C1 = G.shape[2]
    Qc = jnp.dot(newR_ref[0], wa_ref[...], preferred_element_type=F32)
    h = G - Qc.reshape(1, Sb, C1)
    h = jax.nn.relu(h * (g0_ref[...].reshape(1, 1, C1) * BN)
                    + b0_ref[...].reshape(1, 1, C1))
    h = h.reshape(K * Sb, C1)
    h = jax.nn.relu(jnp.dot(h, w1_ref[...], preferred_element_type=F32)
                    * (g1_ref[...] * BN) + b1_ref[...])
    h = jax.nn.relu(jnp.dot(h, w2_ref[...], preferred_element_type=F32)
                    * (g2_ref[...] * BN) + b2_ref[...])
    C2 = h.shape[1]
    out_ref[0] = jnp.max(h.reshape(K, Sb, C2), axis=0)


def _mlp(G, newR, waT, g0, b0, w1T, g1, b1, w2T, g2, b2):
    B, K, S, C1 = G.shape
    C1b = w1T.shape[1]
    C2 = w2T.shape[1]
    Sb = min(S, 256)
    return pl.pallas_call(
        functools.partial(_mlp_body, K),
        grid=(B, S // Sb),
        in_specs=[
            pl.BlockSpec((1, K, Sb, C1), lambda b, j: (b, 0, j, 0)),
            pl.BlockSpec((1, Sb, 8), lambda b, j: (b, j, 0)),
            pl.BlockSpec((8, C1), lambda b, j: (0, 0)),
            pl.BlockSpec((1, C1), lambda b, j: (0, 0)),
            pl.BlockSpec((1, C1), lambda b, j: (0, 0)),
            pl.BlockSpec((C1, C1b), lambda b, j: (0, 0)),
            pl.BlockSpec((1, C1b), lambda b, j: (0, 0)),
            pl.BlockSpec((1, C1b), lambda b, j: (0, 0)),
            pl.BlockSpec((C1b, C2), lambda b, j: (0, 0)),
            pl.BlockSpec((1, C2), lambda b, j: (0, 0)),
            pl.BlockSpec((1, C2), lambda b, j: (0, 0)),
        ],
        out_specs=pl.BlockSpec((1, Sb, C2), lambda b, j: (b, j, 0)),
        out_shape=jax.ShapeDtypeStruct((B, S, C2), F32),
    )(G, newR, waT, g0.reshape(1, -1), b0.reshape(1, -1), w1T,
      g1.reshape(1, -1), b1.reshape(1, -1), w2T, g2.reshape(1, -1),
      b2.reshape(1, -1))


def _fp_body(S2, nl, x1_ref, x2_ref, p1_ref, p2_ref, *refs):
    wa_ref, wb_ref = refs[0], refs[1]
    lrefs = refs[2:-1]
    out_ref = refs[-1]
    q1 = x1_ref[0]
    q2 = x2_ref[0]
    Sb = q1.shape[0]
    qn1 = jnp.sum(q1 * q1, axis=1, keepdims=True)
    qn2 = jnp.sum(q2 * q2, axis=0, keepdims=True)
    d = (qn1 + qn2) - 2.0 * jnp.dot(q1, q2, preferred_element_type=F32)
    iota = lax.broadcasted_iota(I32, (Sb, S2), 1)
    ws, idxs = [], []
    for _ in range(3):
        dk = jnp.min(d, axis=1, keepdims=True)
        ik = jnp.min(jnp.where(d == dk, iota, S2), axis=1, keepdims=True)
        d = jnp.where(iota == ik, jnp.float32(jnp.inf), d)
        ws.append(1.0 / (jnp.maximum(dk, 0.0) + 1e-8))
        idxs.append(ik)
    wsum = (ws[0] + ws[1]) + ws[2]
    wfull = jnp.where(iota == idxs[0], ws[0] / wsum, 0.0)
    wfull = wfull + jnp.where(iota == idxs[1], ws[1] / wsum, 0.0)
    wfull = wfull + jnp.where(iota == idxs[2], ws[2] / wsum, 0.0)
    T2 = jnp.dot(p2_ref[0], wb_ref[...], preferred_element_type=F32)
    pre = jnp.dot(wfull, T2, preferred_element_type=F32)
    pre = pre + jnp.dot(p1_ref[0], wa_ref[...], preferred_element_type=F32)
    g0, b0 = lrefs[0], lrefs[1]
    h = jax.nn.relu(pre * (g0[...] * BN) + b0[...])
    i = 2
    for _ in range(nl - 1):
        w_, g_, b_ = lrefs[i], lrefs[i + 1], lrefs[i + 2]
        h = jax.nn.relu(jnp.dot(h, w_[...], preferred_element_type=F32)
                        * (g_[...] * BN) + b_[...])
        i += 3
    if len(lrefs) > i:
        w_, g_, b_ = lrefs[i], lrefs[i + 1], lrefs[i + 2]
        h = jax.nn.relu(jnp.dot(h, w_[...], preferred_element_type=F32)
                        * (g_[...] * BN) + b_[...])
        w_, b_ = lrefs[i + 3], lrefs[i + 4]
        h = jnp.dot(h, w_[...], preferred_element_type=F32) + b_[...]
    out_ref[0] = h


def _fp(x1R, x2T, p1R, p2R, waT, wbT, layers, fc=None):
    B, S1, _ = x1R.shape
    S2 = x2T.shape[2]
    C1p = p1R.shape[2]
    C2p = p2R.shape[2]
    Sb = min(S1, 512)
    nl = len(layers)
    flat = [layers[0][1].reshape(1, -1), layers[0][2].reshape(1, -1)]
    for (wT, g, b) in layers[1:]:
        flat += [wT, g.reshape(1, -1), b.reshape(1, -1)]
    if fc is not None:
        fc1wT, fc1g, fc1b, fc2wT, fc2bias = fc
        flat += [fc1wT, fc1g.reshape(1, -1), fc1b.reshape(1, -1), fc2wT,
                 fc2bias.reshape(1, -1)]
    especs = [pl.BlockSpec(a.shape, lambda b, j, n=a.ndim: (0,) * n)
              for a in flat]
    Cout = fc[3].shape[1] if fc is not None else layers[-1][0].shape[1]
    return pl.pallas_call(
        functools.partial(_fp_body, S2, nl),
        grid=(B, S1 // Sb),
        in_specs=[
            pl.BlockSpec((1, Sb, 8), lambda b, j: (b, j, 0)),
            pl.BlockSpec((1, 8, S2), lambda b, j: (b, 0, 0)),
            pl.BlockSpec((1, Sb, C1p), lambda b, j: (b, j, 0)),
            pl.BlockSpec((1, S2, C2p), lambda b, j: (b, 0, 0)),
            pl.BlockSpec(waT.shape, lambda b, j: (0, 0)),
            pl.BlockSpec(wbT.shape, lambda b, j: (0, 0)),
        ] + especs,
        out_specs=pl.BlockSpec((1, Sb, Cout), lambda b, j: (b, j, 0)),
        out_shape=jax.ShapeDtypeStruct((B, S1, Cout), F32),
    )(x1R, x2T, p1R, p2R, waT, wbT, *flat)


def _sa_feat(xyzR, gidx, newR, ptsR, params, prefix, nsample):
    B, N, _ = xyzR.shape
    npoint = newR.shape[1]
    w0 = params[prefix + '_w0']
    C1 = w0.shape[0]
    waT = jnp.zeros((8, C1), F32).at[:3].set(w0[:, :3].T)
    wbT = w0[:, 3:].T
    PW = _pw(xyzR, ptsR, waT, wbT)
    G = _sc_gather(PW.reshape(B * N, C1), gidx.reshape(-1))
    G = G.reshape(B, nsample, npoint, C1)
    return _mlp(G, newR, waT, params[prefix + '_g0'], params[prefix + '_b0'],
                params[prefix + '_w1'].T, params[prefix + '_g1'],
                params[prefix + '_b1'], params[prefix + '_w2'].T,
                params[prefix + '_g2'], params[prefix + '_b2'])


def _fp_level(x1R, x2T, p1R, p2R, params, prefix, nlayers, fc=None):
    C1p = p1R.shape[2]
    w0 = params[prefix + '_w0']
    waT = w0[:, :C1p].T
    if C1p < 8:
        waT = jnp.zeros((8, w0.shape[0]), F32).at[:C1p].set(waT)
        B, S1, _ = p1R.shape
        p1R = jnp.concatenate([p1R, jnp.zeros((B, S1, 8 - C1p), F32)], axis=-1)
    wbT = w0[:, C1p:].T
    layers = [(waT, params[prefix + '_g0'], params[prefix + '_b0'])]
    for j in range(1, nlayers):
        layers.append((params[prefix + '_w' + str(j)].T,
                       params[prefix + '_g' + str(j)],
                       params[prefix + '_b' + str(j)]))
    return _fp(x1R, x2T, p1R, p2R, waT, wbT, layers, fc=fc)


def kernel(xyz, points, params):
    B, N, _ = xyz.shape
    SA = (('sa0', 1024, 0.1), ('sa1', 256, 0.2), ('sa2', 64, 0.4),
          ('sa3', 16, 0.8))
    xyzR = jnp.concatenate([xyz, jnp.zeros((B, N, 5), F32)], axis=-1)
    xs = [xyzR]
    gs = []
    cur = xyzR
    for (prefix, npoint, radius) in SA:
        n = cur.shape[1]
        curT = jnp.transpose(cur, (0, 2, 1))
        newR = _fps(curT[:, :3, :].reshape(B, 3, 8, n // 8), cur,
                    cur[:, :, :3].reshape(-1), npoint)
        gs.append(_bq(cur, jnp.transpose(newR, (0, 2, 1)), radius, 32))
        xs.append(newR)
        cur = newR
    feats = [points]
    for i, (prefix, npoint, radius) in enumerate(SA):
        feats.append(_sa_feat(xs[i], gs[i], xs[i + 1], feats[i], params,
                              prefix, 32))
    l1p, l2p, l3p, l4p = feats[1], feats[2], feats[3], feats[4]
    x1T = jnp.transpose(xs[1], (0, 2, 1))
    x2T = jnp.transpose(xs[2], (0, 2, 1))
    x3T = jnp.transpose(xs[3], (0, 2, 1))
    x4T = jnp.transpose(xs[4], (0, 2, 1))
    l3p = _fp_level(xs[3], x4T, l3p, l4p, params, 'fp0', 2)
    l2p = _fp_level(xs[2], x3T, l2p, l3p, params, 'fp1', 2)
    l1p = _fp_level(xs[1], x2T, l1p, l2p, params, 'fp2', 2)
    fc = (params['fc1_w'].T, params['fc1_g'], params['fc1_b'],
          params['fc2_w'].T, params['fc2_bias'])
    return _fp_level(xyzR, x1T, points, l1p, params, 'fp3', 3, fc=fc)
